# Initial kernel scaffold; baseline (speedup 1.0000x reference)
import jax
import jax.numpy as jnp
from jax.experimental import pallas as pl


def kernel(x, edge_index, W_pre1, b_pre1, W_pre2, b_pre2, W_ih, b_ih, W_hh, b_hh, prev0, W_conv1, b_conv1, W_conv2, b_conv2, W_post, b_post):
    raise NotImplementedError("write your pallas kernel here")


# speedup vs baseline: 9.2696x; 9.2696x over previous
import functools

import jax
import jax.numpy as jnp
from jax import lax
from jax.experimental import pallas as pl
from jax.experimental.pallas import tpu as pltpu
from jax.experimental.pallas import tpu_sc as plsc

N = 10000
H = 128
E = 640000
NC, NS = 2, 16
NW = NC * NS
EPW = E // NW
K = 80
NCHUNK = EPW // K
NP = 10240
RPT = NP // NS

PASSES = 3
NH = 3360
AROW = 3456
APT = AROW // NS
TRASH = 3400

R = 2000
GRID = N // R

_sc_mesh = plsc.VectorSubcoreMesh(core_axis_name="c", subcore_axis_name="s")


@functools.partial(
    pl.kernel,
    out_type=jax.ShapeDtypeStruct((NC, NS, NP), jnp.float32),
    mesh=_sc_mesh,
    compiler_params=pltpu.CompilerParams(needs_layout_passes=False),
    scratch_types=[
        pltpu.VMEM((EPW,), jnp.int32),
        pltpu.VMEM((NP,), jnp.float32),
    ],
)
def _sc_degree(dst_hbm, out_hbm, dst_v, hist_v):
    c = lax.axis_index("c")
    s = lax.axis_index("s")
    wid = s * NC + c
    pltpu.sync_copy(dst_hbm.at[wid], dst_v)
    def zfill(i, _):
        hist_v[pl.ds(i * 16, 16)] = jnp.zeros((16,), jnp.float32)
        return 0
    lax.fori_loop(0, NP // 16, zfill, 0)
    ones = jnp.ones((16,), jnp.float32)

    def body(i, _):
        idx = dst_v[pl.ds(i * 16, 16)]
        plsc.addupdate_scatter(hist_v, [idx], ones)
        return 0
    lax.fori_loop(0, EPW // 16, body, 0)
    pltpu.sync_copy(hist_v, out_hbm.at[c, s])


@functools.partial(
    pl.kernel,
    out_type=jax.ShapeDtypeStruct((NC, PASSES, AROW, H), jnp.float32),
    mesh=_sc_mesh,
    scratch_types=[
        pltpu.VMEM((NCHUNK, K), jnp.int32),
        pltpu.VMEM((NCHUNK, K), jnp.int32),
        pltpu.VMEM((2, K, H), jnp.float32),
        pltpu.VMEM((128, H), jnp.float32),
        pltpu.SemaphoreType.DMA,
        pltpu.SemaphoreType.DMA,
        pltpu.VMEM_SHARED((AROW, H), jnp.float32),
    ],
)
def _sc_propagate(m_hbm, src_hbm, dstl_hbm, out_hbm,
                  src_v, dstl_v, rows_v, zrows_v, sem0, sem1, acc):
    c = lax.axis_index("c")
    s = lax.axis_index("s")
    wid = s * NC + c
    pltpu.sync_copy(src_hbm.at[wid], src_v)

    def zrow(i, _):
        for j in range(H // 16):
            zrows_v[i, pl.ds(j * 16, 16)] = jnp.zeros((16,), jnp.float32)
        return 0
    lax.fori_loop(0, 128, zrow, 0)

    for half in range(PASSES):
        pltpu.sync_copy(dstl_hbm.at[half, wid], dstl_v)
        pltpu.sync_copy(zrows_v, acc.at[pl.ds(s * APT, 128)])
        pltpu.sync_copy(zrows_v.at[pl.ds(0, APT - 128)],
                        acc.at[pl.ds(s * APT + 128, APT - 128)])
        plsc.subcore_barrier()

        def body(g2, _):
            d0 = pltpu.async_copy(m_hbm.at[src_v.at[2 * g2]], rows_v.at[0],
                                  sem0)
            d1 = pltpu.async_copy(m_hbm.at[src_v.at[2 * g2 + 1]],
                                  rows_v.at[1], sem1)
            d0.wait()
            pltpu.sync_copy(rows_v.at[0], acc.at[dstl_v.at[2 * g2]], add=True)
            d1.wait()
            pltpu.sync_copy(rows_v.at[1], acc.at[dstl_v.at[2 * g2 + 1]],
                            add=True)
            return 0
        lax.fori_loop(0, NCHUNK // 2, body, 0)

        plsc.subcore_barrier()
        pltpu.sync_copy(acc.at[pl.ds(s * APT, APT)],
                        out_hbm.at[c, half, pl.ds(s * APT, APT)])


def _lrelu(v):
    return jnp.where(v >= 0, v, 0.01 * v)


def _prep_body(deg_ref, dst_ref, dinv_ref, *dl_refs):
    d = jnp.sum(deg_ref[...], axis=0, keepdims=True) + 1.0
    dinv_ref[...] = jnp.transpose(lax.rsqrt(d), (1, 0))
    dv = dst_ref[...]
    for p in range(PASSES):
        lp = dv - p * NH
        dl_refs[p][...] = jnp.where((lp < 0) | (lp >= NH), TRASH, lp)


def _prologue_body(x_ref, p0_ref, dinv_ref, w1_ref, b1_ref, w2_ref, b2_ref,
                   wih_ref, bih_ref, whh_ref, bhh_ref, wc1_ref,
                   emb0_ref, m1_ref):
    f32 = jnp.float32
    h1 = _lrelu(jnp.dot(x_ref[...], w1_ref[...], preferred_element_type=f32)
                + b1_ref[...])
    h2 = _lrelu(jnp.dot(h1, w2_ref[...], preferred_element_type=f32)
                + b2_ref[...])
    gi = jnp.dot(h2, wih_ref[...], preferred_element_type=f32) + bih_ref[...]
    p0 = p0_ref[...]
    gh = jnp.dot(p0, whh_ref[...], preferred_element_type=f32) + bhh_ref[...]
    r = jax.nn.sigmoid(gi[:, :H] + gh[:, :H])
    z = jax.nn.sigmoid(gi[:, H:2 * H] + gh[:, H:2 * H])
    n = jnp.tanh(gi[:, 2 * H:] + r * gh[:, 2 * H:])
    e0 = (1.0 - z) * n + z * p0
    emb0_ref[...] = e0
    m1_ref[...] = dinv_ref[...] * jnp.dot(e0, wc1_ref[...],
                                          preferred_element_type=f32)


def _combine_body(p_ref, m_ref, dinv_ref, b_ref, w_ref, emb_ref, nxt_ref):
    dinv = dinv_ref[...]
    conv = dinv * (p_ref[0] + p_ref[1] + m_ref[...]) + b_ref[...]
    e = _lrelu(conv)
    emb_ref[...] = e
    nxt_ref[...] = dinv * jnp.dot(e, w_ref[...],
                                  preferred_element_type=jnp.float32)


def _head_body(p_ref, m_ref, dinv_ref, b_ref, w_ref, bp_ref,
               emb_ref, out_ref):
    conv = dinv_ref[...] * (p_ref[0] + p_ref[1] + m_ref[...]) + b_ref[...]
    e = _lrelu(conv)
    emb_ref[...] = e
    out_ref[...] = (jnp.dot(e, w_ref[...], preferred_element_type=jnp.float32)
                    + bp_ref[...])


def _row_spec(cols):
    return pl.BlockSpec((R, cols), lambda i: (i, 0))


def _full_spec(shape):
    nd = len(shape)
    return pl.BlockSpec(shape, lambda i: (0,) * nd)


_PART_SPEC = pl.BlockSpec((NC, R, H), lambda i: (0, i, 0))
_DINV_SPEC = pl.BlockSpec((R, 1), lambda i: (i, 0))


def kernel(x, edge_index, W_pre1, b_pre1, W_pre2, b_pre2, W_ih, b_ih,
           W_hh, b_hh, prev0, W_conv1, b_conv1, W_conv2, b_conv2,
           W_post, b_post):
    ei = edge_index.astype(jnp.int32)
    src = ei[0].reshape(NW, NCHUNK, K)
    dst = ei[1].reshape(NW, NCHUNK, K)

    dst_flat = ei[1].reshape(NW, EPW)
    deg2 = _sc_degree(dst_flat).reshape(NW, NP)

    prep = pl.pallas_call(
        _prep_body,
        out_shape=[jax.ShapeDtypeStruct((NP, 1), jnp.float32)]
        + [jax.ShapeDtypeStruct((NW, NCHUNK, K), jnp.int32)] * PASSES,
    )(deg2, dst)
    dinv = prep[0]
    dstl = jnp.stack(prep[1:], axis=0)

    row2 = lambda v: v.reshape(1, -1)
    emb0, m1 = pl.pallas_call(
        _prologue_body,
        grid=(GRID,),
        in_specs=[
            _row_spec(128), _row_spec(H), _DINV_SPEC,
            _full_spec((128, 256)), _full_spec((1, 256)),
            _full_spec((256, H)), _full_spec((1, H)),
            _full_spec((H, 3 * H)), _full_spec((1, 3 * H)),
            _full_spec((H, 3 * H)), _full_spec((1, 3 * H)),
            _full_spec((H, H)),
        ],
        out_specs=[_row_spec(H), _row_spec(H)],
        out_shape=[jax.ShapeDtypeStruct((N, H), jnp.float32),
                   jax.ShapeDtypeStruct((N, H), jnp.float32)],
    )(x, prev0, dinv, W_pre1, row2(b_pre1), W_pre2, row2(b_pre2),
      W_ih.T, row2(b_ih), W_hh.T, row2(b_hh), W_conv1)

    def propagate(m):
        p = _sc_propagate(m, src, dstl)
        parts = [p[:, i, :NH] for i in range(PASSES)]
        parts[-1] = parts[-1][:, :N - (PASSES - 1) * NH]
        return jnp.concatenate(parts, axis=1)

    p1 = propagate(m1)
    emb1, m2 = pl.pallas_call(
        _combine_body,
        grid=(GRID,),
        in_specs=[_PART_SPEC, _row_spec(H), _DINV_SPEC,
                  _full_spec((1, H)), _full_spec((H, H))],
        out_specs=[_row_spec(H), _row_spec(H)],
        out_shape=[jax.ShapeDtypeStruct((N, H), jnp.float32),
                   jax.ShapeDtypeStruct((N, H), jnp.float32)],
    )(p1, m1, dinv, row2(b_conv1), W_conv2)

    p2 = propagate(m2)
    emb2, out = pl.pallas_call(
        _head_body,
        grid=(GRID,),
        in_specs=[_PART_SPEC, _row_spec(H), _DINV_SPEC,
                  _full_spec((1, H)), _full_spec((H, 64)),
                  _full_spec((1, 64))],
        out_specs=[_row_spec(H), _row_spec(64)],
        out_shape=[jax.ShapeDtypeStruct((N, H), jnp.float32),
                   jax.ShapeDtypeStruct((N, 64), jnp.float32)],
    )(p2, m2, dinv, row2(b_conv2), W_post, row2(b_post))

    return (out, emb0, emb1, emb2)


# speedup vs baseline: 9.5306x; 1.0282x over previous
import functools

import jax
import jax.numpy as jnp
from jax import lax
from jax.experimental import pallas as pl
from jax.experimental.pallas import tpu as pltpu
from jax.experimental.pallas import tpu_sc as plsc

N = 10000
H = 128
E = 640000
NC, NS = 2, 16
NW = NC * NS
K = 125
NCHUNK = 160
EPW = NCHUNK * K
NP = 10240
RPT = NP // NS

PASSES = 3
NH = 3360
AROW = 3456
APT = AROW // NS
TRASH = 3400

R = 2000
GRID = N // R

_sc_mesh = plsc.VectorSubcoreMesh(core_axis_name="c", subcore_axis_name="s")


@functools.partial(
    pl.kernel,
    out_type=jax.ShapeDtypeStruct((NC, NS, NP), jnp.float32),
    mesh=_sc_mesh,
    compiler_params=pltpu.CompilerParams(needs_layout_passes=False),
    scratch_types=[
        pltpu.VMEM((EPW,), jnp.int32),
        pltpu.VMEM((NP,), jnp.float32),
    ],
)
def _sc_degree(dst_hbm, out_hbm, dst_v, hist_v):
    c = lax.axis_index("c")
    s = lax.axis_index("s")
    wid = s * NC + c
    pltpu.sync_copy(dst_hbm.at[wid], dst_v)
    def zfill(i, _):
        hist_v[pl.ds(i * 16, 16)] = jnp.zeros((16,), jnp.float32)
        return 0
    lax.fori_loop(0, NP // 16, zfill, 0)
    ones = jnp.ones((16,), jnp.float32)

    def body(i, _):
        idx = dst_v[pl.ds(i * 16, 16)]
        plsc.addupdate_scatter(hist_v, [idx], ones)
        return 0
    lax.fori_loop(0, EPW // 16, body, 0)
    pltpu.sync_copy(hist_v, out_hbm.at[c, s])


@functools.partial(
    pl.kernel,
    out_type=jax.ShapeDtypeStruct((NC, PASSES, AROW, H), jnp.float32),
    mesh=_sc_mesh,
    scratch_types=[
        pltpu.VMEM((NCHUNK, K), jnp.int32),
        pltpu.VMEM((NCHUNK, K), jnp.int32),
        pltpu.VMEM((2, K, H), jnp.float32),
        pltpu.VMEM((64, H), jnp.float32),
        pltpu.SemaphoreType.DMA,
        pltpu.SemaphoreType.DMA,
        pltpu.SemaphoreType.DMA,
        pltpu.SemaphoreType.DMA,
        pltpu.VMEM_SHARED((AROW, H), jnp.float32),
    ],
)
def _sc_propagate(m_hbm, src_hbm, dstl_hbm, out_hbm,
                  src_v, dstl_v, rows_v, zrows_v, sem0, sem1, sem2, sem3,
                  acc):
    c = lax.axis_index("c")
    s = lax.axis_index("s")
    wid = s * NC + c
    pltpu.sync_copy(src_hbm.at[wid], src_v)

    def zrow(i, _):
        for j in range(H // 16):
            zrows_v[i, pl.ds(j * 16, 16)] = jnp.zeros((16,), jnp.float32)
        return 0
    lax.fori_loop(0, 64, zrow, 0)

    def gather(g, b):
        sem = sem0 if b == 0 else sem1
        return pltpu.async_copy(m_hbm.at[src_v.at[g]], rows_v.at[b], sem)

    def scatter(g, b):
        sem = sem2 if b == 0 else sem3
        return pltpu.async_copy(rows_v.at[b], acc.at[dstl_v.at[g]], sem,
                                add=True)

    def wait_gather(g, b):
        sem = sem0 if b == 0 else sem1
        pltpu.make_async_copy(m_hbm.at[src_v.at[g]], rows_v.at[b], sem).wait()

    def wait_scatter(b):
        sem = sem2 if b == 0 else sem3
        pltpu.make_async_copy(rows_v.at[b], acc.at[dstl_v.at[0]], sem).wait()

    for half in range(PASSES):
        pltpu.sync_copy(dstl_hbm.at[half, wid], dstl_v)
        for t in range(3):
            pltpu.sync_copy(zrows_v, acc.at[pl.ds(s * APT + 64 * t, 64)])
        pltpu.sync_copy(zrows_v.at[pl.ds(0, APT - 192)],
                        acc.at[pl.ds(s * APT + 192, APT - 192)])
        plsc.subcore_barrier()

        gather(0, 0)
        gather(1, 1)

        def body(q, _):
            wait_gather(2 * q, 0)
            scatter(2 * q, 0)
            wait_gather(2 * q + 1, 1)
            scatter(2 * q + 1, 1)
            wait_scatter(0)
            gather(2 * q + 2, 0)
            wait_scatter(1)
            gather(2 * q + 3, 1)
            return 0
        lax.fori_loop(0, NCHUNK // 2 - 1, body, 0)

        last = NCHUNK - 2
        wait_gather(last, 0)
        scatter(last, 0)
        wait_gather(last + 1, 1)
        scatter(last + 1, 1)
        wait_scatter(0)
        wait_scatter(1)

        plsc.subcore_barrier()
        pltpu.sync_copy(acc.at[pl.ds(s * APT, APT)],
                        out_hbm.at[c, half, pl.ds(s * APT, APT)])


def _lrelu(v):
    return jnp.where(v >= 0, v, 0.01 * v)


def _prep_body(deg_ref, dst_ref, dinv_ref, *dl_refs):
    d = jnp.sum(deg_ref[...], axis=0, keepdims=True) + 1.0
    dinv_ref[...] = jnp.transpose(lax.rsqrt(d), (1, 0))
    dv = dst_ref[...]
    for p in range(PASSES):
        lp = dv - p * NH
        dl_refs[p][...] = jnp.where((lp < 0) | (lp >= NH), TRASH, lp)


def _prologue_body(x_ref, p0_ref, dinv_ref, w1_ref, b1_ref, w2_ref, b2_ref,
                   wih_ref, bih_ref, whh_ref, bhh_ref, wc1_ref,
                   emb0_ref, m1_ref):
    f32 = jnp.float32
    h1 = _lrelu(jnp.dot(x_ref[...], w1_ref[...], preferred_element_type=f32)
                + b1_ref[...])
    h2 = _lrelu(jnp.dot(h1, w2_ref[...], preferred_element_type=f32)
                + b2_ref[...])
    gi = jnp.dot(h2, wih_ref[...], preferred_element_type=f32) + bih_ref[...]
    p0 = p0_ref[...]
    gh = jnp.dot(p0, whh_ref[...], preferred_element_type=f32) + bhh_ref[...]
    r = jax.nn.sigmoid(gi[:, :H] + gh[:, :H])
    z = jax.nn.sigmoid(gi[:, H:2 * H] + gh[:, H:2 * H])
    n = jnp.tanh(gi[:, 2 * H:] + r * gh[:, 2 * H:])
    e0 = (1.0 - z) * n + z * p0
    emb0_ref[...] = e0
    m1_ref[...] = dinv_ref[...] * jnp.dot(e0, wc1_ref[...],
                                          preferred_element_type=f32)


def _combine_body(p_ref, m_ref, dinv_ref, b_ref, w_ref, emb_ref, nxt_ref):
    dinv = dinv_ref[...]
    conv = dinv * (p_ref[0] + p_ref[1] + m_ref[...]) + b_ref[...]
    e = _lrelu(conv)
    emb_ref[...] = e
    nxt_ref[...] = dinv * jnp.dot(e, w_ref[...],
                                  preferred_element_type=jnp.float32)


def _head_body(p_ref, m_ref, dinv_ref, b_ref, w_ref, bp_ref,
               emb_ref, out_ref):
    conv = dinv_ref[...] * (p_ref[0] + p_ref[1] + m_ref[...]) + b_ref[...]
    e = _lrelu(conv)
    emb_ref[...] = e
    out_ref[...] = (jnp.dot(e, w_ref[...], preferred_element_type=jnp.float32)
                    + bp_ref[...])


def _row_spec(cols):
    return pl.BlockSpec((R, cols), lambda i: (i, 0))


def _full_spec(shape):
    nd = len(shape)
    return pl.BlockSpec(shape, lambda i: (0,) * nd)


_PART_SPEC = pl.BlockSpec((NC, R, H), lambda i: (0, i, 0))
_DINV_SPEC = pl.BlockSpec((R, 1), lambda i: (i, 0))


def kernel(x, edge_index, W_pre1, b_pre1, W_pre2, b_pre2, W_ih, b_ih,
           W_hh, b_hh, prev0, W_conv1, b_conv1, W_conv2, b_conv2,
           W_post, b_post):
    ei = edge_index.astype(jnp.int32)
    src = ei[0].reshape(NW, NCHUNK, K)
    dst = ei[1].reshape(NW, NCHUNK, K)

    deg2 = _sc_degree(ei[1].reshape(NW, EPW)).reshape(NW, NP)

    prep = pl.pallas_call(
        _prep_body,
        out_shape=[jax.ShapeDtypeStruct((NP, 1), jnp.float32)]
        + [jax.ShapeDtypeStruct((NW, NCHUNK, K), jnp.int32)] * PASSES,
    )(deg2, dst)
    dinv = prep[0]
    dstl = jnp.stack(prep[1:], axis=0)

    row2 = lambda v: v.reshape(1, -1)
    emb0, m1 = pl.pallas_call(
        _prologue_body,
        grid=(GRID,),
        in_specs=[
            _row_spec(128), _row_spec(H), _DINV_SPEC,
            _full_spec((128, 256)), _full_spec((1, 256)),
            _full_spec((256, H)), _full_spec((1, H)),
            _full_spec((H, 3 * H)), _full_spec((1, 3 * H)),
            _full_spec((H, 3 * H)), _full_spec((1, 3 * H)),
            _full_spec((H, H)),
        ],
        out_specs=[_row_spec(H), _row_spec(H)],
        out_shape=[jax.ShapeDtypeStruct((N, H), jnp.float32),
                   jax.ShapeDtypeStruct((N, H), jnp.float32)],
    )(x, prev0, dinv, W_pre1, row2(b_pre1), W_pre2, row2(b_pre2),
      W_ih.T, row2(b_ih), W_hh.T, row2(b_hh), W_conv1)

    def propagate(m):
        p = _sc_propagate(m, src, dstl)
        parts = [p[:, i, :NH] for i in range(PASSES)]
        parts[-1] = parts[-1][:, :N - (PASSES - 1) * NH]
        return jnp.concatenate(parts, axis=1)

    p1 = propagate(m1)
    emb1, m2 = pl.pallas_call(
        _combine_body,
        grid=(GRID,),
        in_specs=[_PART_SPEC, _row_spec(H), _DINV_SPEC,
                  _full_spec((1, H)), _full_spec((H, H))],
        out_specs=[_row_spec(H), _row_spec(H)],
        out_shape=[jax.ShapeDtypeStruct((N, H), jnp.float32),
                   jax.ShapeDtypeStruct((N, H), jnp.float32)],
    )(p1, m1, dinv, row2(b_conv1), W_conv2)

    p2 = propagate(m2)
    emb2, out = pl.pallas_call(
        _head_body,
        grid=(GRID,),
        in_specs=[_PART_SPEC, _row_spec(H), _DINV_SPEC,
                  _full_spec((1, H)), _full_spec((H, 64)),
                  _full_spec((1, 64))],
        out_specs=[_row_spec(H), _row_spec(64)],
        out_shape=[jax.ShapeDtypeStruct((N, H), jnp.float32),
                   jax.ShapeDtypeStruct((N, 64), jnp.float32)],
    )(p2, m2, dinv, row2(b_conv2), W_post, row2(b_post))

    return (out, emb0, emb1, emb2)


# speedup vs baseline: 9.6516x; 1.0127x over previous
import functools

import jax
import jax.numpy as jnp
from jax import lax
from jax.experimental import pallas as pl
from jax.experimental.pallas import tpu as pltpu
from jax.experimental.pallas import tpu_sc as plsc

N = 10000
H = 128
E = 640000
NC, NS = 2, 16
NW = NC * NS
K = 125
NCHUNK = 160
EPW = NCHUNK * K
NP = 10240
RPT = NP // NS

PASSES = 3
NH = 3360
AROW = 3456
APT = AROW // NS
TRASH = 3400

R = 2000
GRID = N // R

_sc_mesh = plsc.VectorSubcoreMesh(core_axis_name="c", subcore_axis_name="s")


@functools.partial(
    pl.kernel,
    out_type=jax.ShapeDtypeStruct((NC, NS, NP), jnp.float32),
    mesh=_sc_mesh,
    compiler_params=pltpu.CompilerParams(needs_layout_passes=False),
    scratch_types=[
        pltpu.VMEM((EPW,), jnp.int32),
        pltpu.VMEM((NP,), jnp.float32),
    ],
)
def _sc_degree(dst_hbm, out_hbm, dst_v, hist_v):
    c = lax.axis_index("c")
    s = lax.axis_index("s")
    wid = s * NC + c
    pltpu.sync_copy(dst_hbm.at[wid], dst_v)
    def zfill(i, _):
        hist_v[pl.ds(i * 16, 16)] = jnp.zeros((16,), jnp.float32)
        return 0
    lax.fori_loop(0, NP // 16, zfill, 0)
    ones = jnp.ones((16,), jnp.float32)

    def body(i, _):
        idx = dst_v[pl.ds(i * 16, 16)]
        plsc.addupdate_scatter(hist_v, [idx], ones)
        return 0
    lax.fori_loop(0, EPW // 16, body, 0)
    pltpu.sync_copy(hist_v, out_hbm.at[c, s])


@functools.partial(
    pl.kernel,
    out_type=jax.ShapeDtypeStruct((NC, PASSES, AROW, H), jnp.float32),
    mesh=_sc_mesh,
    scratch_types=[
        pltpu.VMEM((NCHUNK, K), jnp.int32),
        pltpu.VMEM((NCHUNK, K), jnp.int32),
        pltpu.VMEM((2, K, H), jnp.float32),
        pltpu.VMEM((64, H), jnp.float32),
        pltpu.SemaphoreType.DMA,
        pltpu.SemaphoreType.DMA,
        pltpu.SemaphoreType.DMA,
        pltpu.SemaphoreType.DMA,
        pltpu.VMEM_SHARED((AROW, H), jnp.float32),
    ],
)
def _sc_propagate(m_hbm, src_hbm, dstl_hbm, out_hbm,
                  src_v, dstl_v, rows_v, zrows_v, sem0, sem1, sem2, sem3,
                  acc):
    c = lax.axis_index("c")
    s = lax.axis_index("s")
    wid = s * NC + c
    pltpu.sync_copy(src_hbm.at[wid], src_v)

    def zrow(i, _):
        for j in range(H // 16):
            zrows_v[i, pl.ds(j * 16, 16)] = jnp.zeros((16,), jnp.float32)
        return 0
    lax.fori_loop(0, 64, zrow, 0)

    for half in range(PASSES):
        pltpu.sync_copy(dstl_hbm.at[half, wid], dstl_v)
        for t in range(3):
            pltpu.sync_copy(zrows_v, acc.at[pl.ds(s * APT + 64 * t, 64)])
        pltpu.sync_copy(zrows_v.at[pl.ds(0, APT - 192)],
                        acc.at[pl.ds(s * APT + 192, APT - 192)])
        plsc.subcore_barrier()

        def body(g2, _):
            d0 = pltpu.async_copy(m_hbm.at[src_v.at[2 * g2]], rows_v.at[0],
                                  sem0)
            d1 = pltpu.async_copy(m_hbm.at[src_v.at[2 * g2 + 1]],
                                  rows_v.at[1], sem1)
            d0.wait()
            e0 = pltpu.async_copy(rows_v.at[0], acc.at[dstl_v.at[2 * g2]],
                                  sem2, add=True)
            d1.wait()
            e1 = pltpu.async_copy(rows_v.at[1],
                                  acc.at[dstl_v.at[2 * g2 + 1]],
                                  sem3, add=True)
            e0.wait()
            e1.wait()
            return 0
        lax.fori_loop(0, NCHUNK // 2, body, 0)

        plsc.subcore_barrier()
        pltpu.sync_copy(acc.at[pl.ds(s * APT, APT)],
                        out_hbm.at[c, half, pl.ds(s * APT, APT)])


def _lrelu(v):
    return jnp.where(v >= 0, v, 0.01 * v)


def _prep_body(deg_ref, dst_ref, dinv_ref, *dl_refs):
    d = jnp.sum(deg_ref[...], axis=0, keepdims=True) + 1.0
    dinv_ref[...] = jnp.transpose(lax.rsqrt(d), (1, 0))
    dv = dst_ref[...]
    for p in range(PASSES):
        lp = dv - p * NH
        dl_refs[p][...] = jnp.where((lp < 0) | (lp >= NH), TRASH, lp)


def _prologue_body(x_ref, p0_ref, dinv_ref, w1_ref, b1_ref, w2_ref, b2_ref,
                   wih_ref, bih_ref, whh_ref, bhh_ref, wc1_ref,
                   emb0_ref, m1_ref):
    f32 = jnp.float32
    h1 = _lrelu(jnp.dot(x_ref[...], w1_ref[...], preferred_element_type=f32)
                + b1_ref[...])
    h2 = _lrelu(jnp.dot(h1, w2_ref[...], preferred_element_type=f32)
                + b2_ref[...])
    gi = jnp.dot(h2, wih_ref[...], preferred_element_type=f32) + bih_ref[...]
    p0 = p0_ref[...]
    gh = jnp.dot(p0, whh_ref[...], preferred_element_type=f32) + bhh_ref[...]
    r = jax.nn.sigmoid(gi[:, :H] + gh[:, :H])
    z = jax.nn.sigmoid(gi[:, H:2 * H] + gh[:, H:2 * H])
    n = jnp.tanh(gi[:, 2 * H:] + r * gh[:, 2 * H:])
    e0 = (1.0 - z) * n + z * p0
    emb0_ref[...] = e0
    m1_ref[...] = dinv_ref[...] * jnp.dot(e0, wc1_ref[...],
                                          preferred_element_type=f32)


def _combine_body(p_ref, m_ref, dinv_ref, b_ref, w_ref, emb_ref, nxt_ref):
    dinv = dinv_ref[...]
    conv = dinv * (p_ref[0] + p_ref[1] + m_ref[...]) + b_ref[...]
    e = _lrelu(conv)
    emb_ref[...] = e
    nxt_ref[...] = dinv * jnp.dot(e, w_ref[...],
                                  preferred_element_type=jnp.float32)


def _head_body(p_ref, m_ref, dinv_ref, b_ref, w_ref, bp_ref,
               emb_ref, out_ref):
    conv = dinv_ref[...] * (p_ref[0] + p_ref[1] + m_ref[...]) + b_ref[...]
    e = _lrelu(conv)
    emb_ref[...] = e
    out_ref[...] = (jnp.dot(e, w_ref[...], preferred_element_type=jnp.float32)
                    + bp_ref[...])


def _row_spec(cols):
    return pl.BlockSpec((R, cols), lambda i: (i, 0))


def _full_spec(shape):
    nd = len(shape)
    return pl.BlockSpec(shape, lambda i: (0,) * nd)


_PART_SPEC = pl.BlockSpec((NC, R, H), lambda i: (0, i, 0))
_DINV_SPEC = pl.BlockSpec((R, 1), lambda i: (i, 0))


def kernel(x, edge_index, W_pre1, b_pre1, W_pre2, b_pre2, W_ih, b_ih,
           W_hh, b_hh, prev0, W_conv1, b_conv1, W_conv2, b_conv2,
           W_post, b_post):
    ei = edge_index.astype(jnp.int32)
    src = ei[0].reshape(NW, NCHUNK, K)
    dst = ei[1].reshape(NW, NCHUNK, K)

    deg2 = _sc_degree(ei[1].reshape(NW, EPW)).reshape(NW, NP)

    prep = pl.pallas_call(
        _prep_body,
        out_shape=[jax.ShapeDtypeStruct((NP, 1), jnp.float32)]
        + [jax.ShapeDtypeStruct((NW, NCHUNK, K), jnp.int32)] * PASSES,
    )(deg2, dst)
    dinv = prep[0]
    dstl = jnp.stack(prep[1:], axis=0)

    row2 = lambda v: v.reshape(1, -1)
    emb0, m1 = pl.pallas_call(
        _prologue_body,
        grid=(GRID,),
        in_specs=[
            _row_spec(128), _row_spec(H), _DINV_SPEC,
            _full_spec((128, 256)), _full_spec((1, 256)),
            _full_spec((256, H)), _full_spec((1, H)),
            _full_spec((H, 3 * H)), _full_spec((1, 3 * H)),
            _full_spec((H, 3 * H)), _full_spec((1, 3 * H)),
            _full_spec((H, H)),
        ],
        out_specs=[_row_spec(H), _row_spec(H)],
        out_shape=[jax.ShapeDtypeStruct((N, H), jnp.float32),
                   jax.ShapeDtypeStruct((N, H), jnp.float32)],
    )(x, prev0, dinv, W_pre1, row2(b_pre1), W_pre2, row2(b_pre2),
      W_ih.T, row2(b_ih), W_hh.T, row2(b_hh), W_conv1)

    def propagate(m):
        p = _sc_propagate(m, src, dstl)
        parts = [p[:, i, :NH] for i in range(PASSES)]
        parts[-1] = parts[-1][:, :N - (PASSES - 1) * NH]
        return jnp.concatenate(parts, axis=1)

    p1 = propagate(m1)
    emb1, m2 = pl.pallas_call(
        _combine_body,
        grid=(GRID,),
        in_specs=[_PART_SPEC, _row_spec(H), _DINV_SPEC,
                  _full_spec((1, H)), _full_spec((H, H))],
        out_specs=[_row_spec(H), _row_spec(H)],
        out_shape=[jax.ShapeDtypeStruct((N, H), jnp.float32),
                   jax.ShapeDtypeStruct((N, H), jnp.float32)],
    )(p1, m1, dinv, row2(b_conv1), W_conv2)

    p2 = propagate(m2)
    emb2, out = pl.pallas_call(
        _head_body,
        grid=(GRID,),
        in_specs=[_PART_SPEC, _row_spec(H), _DINV_SPEC,
                  _full_spec((1, H)), _full_spec((H, 64)),
                  _full_spec((1, 64))],
        out_specs=[_row_spec(H), _row_spec(64)],
        out_shape=[jax.ShapeDtypeStruct((N, H), jnp.float32),
                   jax.ShapeDtypeStruct((N, 64), jnp.float32)],
    )(p2, m2, dinv, row2(b_conv2), W_post, row2(b_post))

    return (out, emb0, emb1, emb2)
